# Initial kernel scaffold; baseline (speedup 1.0000x reference)
import jax
import jax.numpy as jnp
from jax.experimental import pallas as pl


def kernel(grid_obs, effect_inds, params):
    raise NotImplementedError("write your pallas kernel here")


# speedup vs baseline: 3.1855x; 3.1855x over previous
import functools

import jax
import jax.numpy as jnp
from jax import lax
from jax.experimental import pallas as pl
from jax.experimental.pallas import tpu as pltpu
from jax.experimental.pallas import tpu_sc as plsc

N_CELLS = 10000
N_EDGES = 320000
OBS = 128
TW = 32
EW = 8
NC, NS = 2, 16
NW = NC * NS
EPW = N_EDGES // NW
CH = 2000
BE = 8000


def _relu(x):
    return jnp.maximum(x, 0.0)


def _dot(a, b):
    return jax.lax.dot(a, b, preferred_element_type=jnp.float32)


def _tables_body(cells, Wab, Wd1, bd1, Wd2, bd2, Wd3, bd3,
                 S1s, S2s, S1d, S2d, tabS, tabD):
    x = cells[...]
    ab = _dot(x, Wab[...])
    h = _relu(_dot(x, Wd1[...]) + bd1[...])
    h = _relu(_dot(h, Wd2[...]) + bd2[...])
    pq = _dot(h, Wd3[...]) + bd3[...]
    tabS[...] = _dot(ab, S1s[...]) + _dot(pq, S2s[...])
    tabD[...] = _dot(ab, S1d[...]) + _dot(pq, S2d[...])


def _sc_gather_body(tabS, tabD, src, dst, gs, gd, idx_v, rows_v, sem):
    wid = lax.axis_index("s") * NC + lax.axis_index("c")
    base = wid * EPW
    for i in range(EPW // CH):
        b = base + i * CH
        for ind, tab, out in ((src, tabS, gs), (dst, tabD, gd)):
            pltpu.sync_copy(ind.at[pl.ds(b, CH)], idx_v)
            pltpu.async_copy(tab.at[idx_v], rows_v, sem).wait()
            pltpu.sync_copy(rows_v, out.at[pl.ds(b, CH)])


def _edge_body(gs, gd, s20, wc32, b1, W2p, b2, w3p, b3, ones8, e8):
    a = gs[...]
    b = gd[...]
    pq = _dot(a, s20[...]) * _dot(b, s20[...])
    h1 = _relu(a + b + _dot(pq, wc32[...]) + b1[...])
    h2 = _relu(_dot(h1, W2p[...]) + b2[...])
    e = _dot(h2, w3p[...]) + b3[...]
    e8[...] = _dot(e, ones8[...])


def _sc_segsum_body(e8, src, zeros_hbm, out, idx_v, val_v, acc_sh, sem):
    c = lax.axis_index("c")
    s = lax.axis_index("s")
    wid = s * NC + c

    @pl.when(s == 0)
    def _():
        pltpu.sync_copy(zeros_hbm, acc_sh)

    plsc.subcore_barrier()
    base = wid * EPW
    for i in range(EPW // CH):
        b = base + i * CH
        pltpu.sync_copy(src.at[pl.ds(b, CH)], idx_v)
        pltpu.sync_copy(e8.at[pl.ds(b, CH)], val_v)
        pltpu.sync_copy(val_v, acc_sh.at[idx_v], add=True)
    plsc.subcore_barrier()

    @pl.when(s == 0)
    def _():
        pltpu.sync_copy(acc_sh, out.at[c])


def _apply_body(cells, tot2, sel0, Wc1, bc1, Wc2, bc2, Wc3, bc3,
                We1, be1, We2, be2, We3, be3,
                Wp1c, wp1t, wp1d, bp1, Wp2, bp2, Wp3, bp3, pred):
    x = cells[...]
    tot = _dot(tot2[0] + tot2[1], sel0[...])
    h = _relu(_dot(x, Wc1[...]) + bc1[...])
    h = _relu(_dot(h, Wc2[...]) + bc2[...])
    adc = _dot(h, Wc3[...]) + bc3[...]
    h = _relu(_dot(tot, We1[...]) + be1[...])
    h = _relu(_dot(h, We2[...]) + be2[...])
    ade = _dot(h, We3[...]) + be3[...]
    g = _relu(_dot(x, Wp1c[...]) + _dot(tot, wp1t[...])
              + _dot(adc * ade, wp1d[...]) + bp1[...])
    g = _relu(_dot(g, Wp2[...]) + bp2[...])
    pred[...] = _dot(g, Wp3[...]) + bp3[...]


def kernel(grid_obs, effect_inds, params):
    cells = grid_obs
    src = effect_inds[0].astype(jnp.int32)
    dst = effect_inds[1].astype(jnp.int32)
    f32 = jnp.float32

    (We1, be1), (We2, be2), (We3, be3) = params['effect']
    Wa, Wb, wc = We1[:OBS], We1[OBS:2 * OBS], We1[2 * OBS]
    edc, edn = params['effect_dotp_cell'], params['effect_dotp_neighbor']
    Wab = jnp.concatenate([Wa, Wb], axis=1)
    Wd1 = jnp.concatenate([edc[0][0], edn[0][0]], axis=1)
    bd1 = jnp.concatenate([edc[0][1], edn[0][1]])[None]
    Wd2 = jnp.zeros((40, 40), f32).at[:20, :20].set(edc[1][0]).at[20:, 20:].set(edn[1][0])
    bd2 = jnp.concatenate([edc[1][1], edn[1][1]])[None]
    Wd3 = jnp.zeros((40, 2), f32).at[:20, 0:1].set(edc[2][0]).at[20:, 1:2].set(edn[2][0])
    bd3 = jnp.concatenate([edc[2][1], edn[2][1]])[None]
    eye20 = jnp.eye(20, dtype=f32)
    S1s = jnp.zeros((40, TW), f32).at[:20, :20].set(eye20)
    S1d = jnp.zeros((40, TW), f32).at[20:, :20].set(eye20)
    S2s = jnp.zeros((2, TW), f32).at[0, 20].set(1.0)
    S2d = jnp.zeros((2, TW), f32).at[1, 20].set(1.0)

    tabS, tabD = pl.pallas_call(
        _tables_body,
        out_shape=(jax.ShapeDtypeStruct((N_CELLS, TW), f32),
                   jax.ShapeDtypeStruct((N_CELLS, TW), f32)),
    )(cells, Wab, Wd1, bd1, Wd2, bd2, Wd3, bd3, S1s, S2s, S1d, S2d)

    mesh = plsc.VectorSubcoreMesh(core_axis_name="c", subcore_axis_name="s")
    gs, gd = pl.kernel(
        _sc_gather_body,
        out_type=(jax.ShapeDtypeStruct((N_EDGES, TW), f32),
                  jax.ShapeDtypeStruct((N_EDGES, TW), f32)),
        mesh=mesh,
        scratch_types=[pltpu.VMEM((CH,), jnp.int32),
                       pltpu.VMEM((CH, TW), f32),
                       pltpu.SemaphoreType.DMA],
        compiler_params=pltpu.CompilerParams(use_tc_tiling_on_sc=False),
    )(tabS, tabD, src, dst)

    s20 = jnp.zeros((TW, 1), f32).at[20, 0].set(1.0)
    wc32 = jnp.zeros((1, TW), f32).at[0, :20].set(wc)
    b1 = jnp.zeros((1, TW), f32).at[0, :20].set(be1)
    W2p = jnp.zeros((TW, TW), f32).at[:20, :20].set(We2)
    b2 = jnp.zeros((1, TW), f32).at[0, :20].set(be2)
    w3p = jnp.zeros((TW, 1), f32).at[:20].set(We3)
    b3 = be3[None]
    ones8 = jnp.ones((1, EW), f32)
    def _w(a):
        return pl.BlockSpec(a.shape, lambda i: (0,) * a.ndim)

    e8 = pl.pallas_call(
        _edge_body,
        grid=(N_EDGES // BE,),
        in_specs=[pl.BlockSpec((BE, TW), lambda i: (i, 0)),
                  pl.BlockSpec((BE, TW), lambda i: (i, 0)),
                  _w(s20), _w(wc32), _w(b1), _w(W2p), _w(b2), _w(w3p),
                  _w(b3), _w(ones8)],
        out_specs=pl.BlockSpec((BE, EW), lambda i: (i, 0)),
        out_shape=jax.ShapeDtypeStruct((N_EDGES, EW), f32),
    )(gs, gd, s20, wc32, b1, W2p, b2, w3p, b3, ones8)

    zeros_cells = jnp.zeros((N_CELLS, EW), f32)
    tot2 = pl.kernel(
        _sc_segsum_body,
        out_type=jax.ShapeDtypeStruct((NC, N_CELLS, EW), f32),
        mesh=mesh,
        scratch_types=[pltpu.VMEM((CH,), jnp.int32),
                       pltpu.VMEM((CH, EW), f32),
                       pltpu.VMEM_SHARED((N_CELLS, EW), f32),
                       pltpu.SemaphoreType.DMA],
        compiler_params=pltpu.CompilerParams(use_tc_tiling_on_sc=False),
    )(e8, src, zeros_cells)

    adc_p, ade_p, app_p = (params['apply_dotp_cell'],
                           params['apply_dotp_effect'], params['apply'])
    (Wp1, bp1), (Wp2, bp2), (Wp3, bp3) = app_p
    sel0 = jnp.zeros((EW, 1), f32).at[0, 0].set(1.0)
    pred = pl.pallas_call(
        _apply_body,
        out_shape=jax.ShapeDtypeStruct((N_CELLS, OBS), f32),
    )(cells, tot2, sel0,
      adc_p[0][0], adc_p[0][1][None], adc_p[1][0], adc_p[1][1][None],
      adc_p[2][0], adc_p[2][1][None],
      ade_p[0][0], ade_p[0][1][None], ade_p[1][0], ade_p[1][1][None],
      ade_p[2][0], ade_p[2][1][None],
      Wp1[:OBS], Wp1[OBS:OBS + 1], Wp1[OBS + 1:OBS + 2], bp1[None],
      Wp2, bp2[None], Wp3, bp3[None])

    return jnp.stack([pred, pred])


# speedup vs baseline: 3.7726x; 1.1843x over previous
import functools

import jax
import jax.numpy as jnp
from jax import lax
from jax.experimental import pallas as pl
from jax.experimental.pallas import tpu as pltpu
from jax.experimental.pallas import tpu_sc as plsc

N_CELLS = 10000
N_EDGES = 320000
OBS = 128
TW = 32
EW = 8
NC, NS = 2, 16
NW = NC * NS
EPW = N_EDGES // NW
CH = 1000
BE = 8000


def _relu(x):
    return jnp.maximum(x, 0.0)


def _dot(a, b):
    return jax.lax.dot(a, b, preferred_element_type=jnp.float32)


def _tables_body(cells, Wab, Wd1, bd1, Wd2, bd2, Wd3, bd3,
                 S1s, S2s, S1d, S2d, tabS, tabD):
    x = cells[...]
    ab = _dot(x, Wab[...])
    h = _relu(_dot(x, Wd1[...]) + bd1[...])
    h = _relu(_dot(h, Wd2[...]) + bd2[...])
    pq = _dot(h, Wd3[...]) + bd3[...]
    tabS[...] = _dot(ab, S1s[...]) + _dot(pq, S2s[...])
    tabD[...] = _dot(ab, S1d[...]) + _dot(pq, S2d[...])


def _sc_gather_body(tabS, tabD, src, dst, gsum,
                    idx_s, idx_d, rows_s, rows_d, sem_s, sem_d):
    wid = lax.axis_index("s") * NC + lax.axis_index("c")
    base = wid * EPW

    def add_rows(i, _):
        for h in range(TW // 16):
            sl = pl.ds(h * 16, 16)
            rows_s[i, sl] = rows_s[i, sl] + rows_d[i, sl]
        return _

    for i in range(EPW // CH):
        b = base + i * CH
        pltpu.sync_copy(src.at[pl.ds(b, CH)], idx_s)
        pltpu.sync_copy(dst.at[pl.ds(b, CH)], idx_d)
        cp_s = pltpu.async_copy(tabS.at[idx_s], rows_s, sem_s)
        cp_d = pltpu.async_copy(tabD.at[idx_d], rows_d, sem_d)
        cp_s.wait()
        cp_d.wait()
        lax.fori_loop(0, CH, add_rows, 0)
        pltpu.sync_copy(rows_s, gsum.at[pl.ds(b, CH)])


def _edge_body(g, s20, s21, wc32, b1, W2p, b2, w3p, b3, ones8, e8):
    a = g[...]
    pq = _dot(a, s20[...]) * _dot(a, s21[...])
    h1 = _relu(a + _dot(pq, wc32[...]) + b1[...])
    h2 = _relu(_dot(h1, W2p[...]) + b2[...])
    e = _dot(h2, w3p[...]) + b3[...]
    e8[...] = _dot(e, ones8[...])


def _sc_segsum_body(e8, src, zeros_hbm, out, idx_v, val_v, acc_sh, sem):
    c = lax.axis_index("c")
    s = lax.axis_index("s")
    wid = s * NC + c

    @pl.when(s == 0)
    def _():
        pltpu.sync_copy(zeros_hbm, acc_sh)

    plsc.subcore_barrier()
    base = wid * EPW
    for i in range(EPW // CH):
        b = base + i * CH
        pltpu.sync_copy(src.at[pl.ds(b, CH)], idx_v)
        pltpu.sync_copy(e8.at[pl.ds(b, CH)], val_v)
        pltpu.sync_copy(val_v, acc_sh.at[idx_v], add=True)
    plsc.subcore_barrier()

    @pl.when(s == 0)
    def _():
        pltpu.sync_copy(acc_sh, out.at[c])


def _apply_body(cells, tot2, sel0, Wc1, bc1, Wc2, bc2, Wc3, bc3,
                We1, be1, We2, be2, We3, be3,
                Wp1c, wp1t, wp1d, bp1, Wp2, bp2, Wp3, bp3, pred):
    x = cells[...]
    tot = _dot(tot2[0] + tot2[1], sel0[...])
    h = _relu(_dot(x, Wc1[...]) + bc1[...])
    h = _relu(_dot(h, Wc2[...]) + bc2[...])
    adc = _dot(h, Wc3[...]) + bc3[...]
    h = _relu(_dot(tot, We1[...]) + be1[...])
    h = _relu(_dot(h, We2[...]) + be2[...])
    ade = _dot(h, We3[...]) + be3[...]
    g = _relu(_dot(x, Wp1c[...]) + _dot(tot, wp1t[...])
              + _dot(adc * ade, wp1d[...]) + bp1[...])
    g = _relu(_dot(g, Wp2[...]) + bp2[...])
    pred[...] = _dot(g, Wp3[...]) + bp3[...]


def kernel(grid_obs, effect_inds, params):
    cells = grid_obs
    src = effect_inds[0].astype(jnp.int32)
    dst = effect_inds[1].astype(jnp.int32)
    f32 = jnp.float32

    (We1, be1), (We2, be2), (We3, be3) = params['effect']
    Wa, Wb, wc = We1[:OBS], We1[OBS:2 * OBS], We1[2 * OBS]
    edc, edn = params['effect_dotp_cell'], params['effect_dotp_neighbor']
    Wab = jnp.concatenate([Wa, Wb], axis=1)
    Wd1 = jnp.concatenate([edc[0][0], edn[0][0]], axis=1)
    bd1 = jnp.concatenate([edc[0][1], edn[0][1]])[None]
    Wd2 = jnp.zeros((40, 40), f32).at[:20, :20].set(edc[1][0]).at[20:, 20:].set(edn[1][0])
    bd2 = jnp.concatenate([edc[1][1], edn[1][1]])[None]
    Wd3 = jnp.zeros((40, 2), f32).at[:20, 0:1].set(edc[2][0]).at[20:, 1:2].set(edn[2][0])
    bd3 = jnp.concatenate([edc[2][1], edn[2][1]])[None]
    eye20 = jnp.eye(20, dtype=f32)
    S1s = jnp.zeros((40, TW), f32).at[:20, :20].set(eye20)
    S1d = jnp.zeros((40, TW), f32).at[20:, :20].set(eye20)
    S2s = jnp.zeros((2, TW), f32).at[0, 20].set(1.0)
    S2d = jnp.zeros((2, TW), f32).at[1, 21].set(1.0)

    tabS, tabD = pl.pallas_call(
        _tables_body,
        out_shape=(jax.ShapeDtypeStruct((N_CELLS, TW), f32),
                   jax.ShapeDtypeStruct((N_CELLS, TW), f32)),
    )(cells, Wab, Wd1, bd1, Wd2, bd2, Wd3, bd3, S1s, S2s, S1d, S2d)

    mesh = plsc.VectorSubcoreMesh(core_axis_name="c", subcore_axis_name="s")
    gsum = pl.kernel(
        _sc_gather_body,
        out_type=jax.ShapeDtypeStruct((N_EDGES, TW), f32),
        mesh=mesh,
        scratch_types=[pltpu.VMEM((CH,), jnp.int32),
                       pltpu.VMEM((CH,), jnp.int32),
                       pltpu.VMEM((CH, TW), f32),
                       pltpu.VMEM((CH, TW), f32),
                       pltpu.SemaphoreType.DMA,
                       pltpu.SemaphoreType.DMA],
        compiler_params=pltpu.CompilerParams(use_tc_tiling_on_sc=False),
    )(tabS, tabD, src, dst)

    s20 = jnp.zeros((TW, 1), f32).at[20, 0].set(1.0)
    s21 = jnp.zeros((TW, 1), f32).at[21, 0].set(1.0)
    wc32 = jnp.zeros((1, TW), f32).at[0, :20].set(wc)
    b1 = jnp.zeros((1, TW), f32).at[0, :20].set(be1)
    W2p = jnp.zeros((TW, TW), f32).at[:20, :20].set(We2)
    b2 = jnp.zeros((1, TW), f32).at[0, :20].set(be2)
    w3p = jnp.zeros((TW, 1), f32).at[:20].set(We3)
    b3 = be3[None]
    ones8 = jnp.ones((1, EW), f32)
    def _w(a):
        return pl.BlockSpec(a.shape, lambda i: (0,) * a.ndim)

    e8 = pl.pallas_call(
        _edge_body,
        grid=(N_EDGES // BE,),
        in_specs=[pl.BlockSpec((BE, TW), lambda i: (i, 0)),
                  _w(s20), _w(s21), _w(wc32), _w(b1), _w(W2p), _w(b2),
                  _w(w3p), _w(b3), _w(ones8)],
        out_specs=pl.BlockSpec((BE, EW), lambda i: (i, 0)),
        out_shape=jax.ShapeDtypeStruct((N_EDGES, EW), f32),
    )(gsum, s20, s21, wc32, b1, W2p, b2, w3p, b3, ones8)

    zeros_cells = jnp.zeros((N_CELLS, EW), f32)
    tot2 = pl.kernel(
        _sc_segsum_body,
        out_type=jax.ShapeDtypeStruct((NC, N_CELLS, EW), f32),
        mesh=mesh,
        scratch_types=[pltpu.VMEM((CH,), jnp.int32),
                       pltpu.VMEM((CH, EW), f32),
                       pltpu.VMEM_SHARED((N_CELLS, EW), f32),
                       pltpu.SemaphoreType.DMA],
        compiler_params=pltpu.CompilerParams(use_tc_tiling_on_sc=False),
    )(e8, src, zeros_cells)

    adc_p, ade_p, app_p = (params['apply_dotp_cell'],
                           params['apply_dotp_effect'], params['apply'])
    (Wp1, bp1), (Wp2, bp2), (Wp3, bp3) = app_p
    sel0 = jnp.zeros((EW, 1), f32).at[0, 0].set(1.0)
    pred = pl.pallas_call(
        _apply_body,
        out_shape=jax.ShapeDtypeStruct((N_CELLS, OBS), f32),
    )(cells, tot2, sel0,
      adc_p[0][0], adc_p[0][1][None], adc_p[1][0], adc_p[1][1][None],
      adc_p[2][0], adc_p[2][1][None],
      ade_p[0][0], ade_p[0][1][None], ade_p[1][0], ade_p[1][1][None],
      ade_p[2][0], ade_p[2][1][None],
      Wp1[:OBS], Wp1[OBS:OBS + 1], Wp1[OBS + 1:OBS + 2], bp1[None],
      Wp2, bp2[None], Wp3, bp3[None])

    return jnp.stack([pred, pred])


# speedup vs baseline: 4.2958x; 1.1387x over previous
import functools

import jax
import jax.numpy as jnp
from jax import lax
from jax.experimental import pallas as pl
from jax.experimental.pallas import tpu as pltpu
from jax.experimental.pallas import tpu_sc as plsc

N_CELLS = 10000
N_EDGES = 320000
OBS = 128
TW = 32
NWIN = 79
NC, NS = 2, 16
NW = NC * NS
EPW = N_EDGES // NW
CH = 1000
BE = 8000


def _relu(x):
    return jnp.maximum(x, 0.0)


def _dot(a, b):
    return jax.lax.dot(a, b, preferred_element_type=jnp.float32)


def _tables_body(cells, Wab, Wd1, bd1, Wd2, bd2, Wd3, bd3,
                 S1s, S2s, S1d, S2d, tabS, tabD):
    x = cells[...]
    ab = _dot(x, Wab[...])
    h = _relu(_dot(x, Wd1[...]) + bd1[...])
    h = _relu(_dot(h, Wd2[...]) + bd2[...])
    pq = _dot(h, Wd3[...]) + bd3[...]
    tabS[...] = _dot(ab, S1s[...]) + _dot(pq, S2s[...])
    tabD[...] = _dot(ab, S1d[...]) + _dot(pq, S2d[...])


def _sc_gather_body(tabS, tabD, src, dst, gsum,
                    idx_s, idx_d, rows_s, rows_d, sem_s, sem_d):
    wid = lax.axis_index("s") * NC + lax.axis_index("c")
    base = wid * EPW

    def add_rows(i, _):
        for h in range(TW // 16):
            sl = pl.ds(h * 16, 16)
            rows_s[i, sl] = rows_s[i, sl] + rows_d[i, sl]
        return _

    for i in range(EPW // CH):
        b = base + i * CH
        pltpu.sync_copy(src.at[pl.ds(b, CH)], idx_s)
        pltpu.sync_copy(dst.at[pl.ds(b, CH)], idx_d)
        cp_s = pltpu.async_copy(tabS.at[idx_s], rows_s, sem_s)
        cp_d = pltpu.async_copy(tabD.at[idx_d], rows_d, sem_d)
        cp_s.wait()
        cp_d.wait()
        lax.fori_loop(0, CH, add_rows, 0)
        pltpu.sync_copy(rows_s, gsum.at[pl.ds(b, CH)])


def _edge_body(wlo, whi, g, src3, s20, s21, wc32, b1, W2p, b2, w3p, b3, tot):
    i = pl.program_id(0)

    @pl.when(i == 0)
    def _():
        tot[...] = jnp.zeros_like(tot)

    a = g[...]
    pq = _dot(a, s20[...]) * _dot(a, s21[...])
    h1 = _relu(a + _dot(pq, wc32[...]) + b1[...])
    h2 = _relu(_dot(h1, W2p[...]) + b2[...])
    e = _dot(h2, w3p[...]) + b3[...]

    srcrow = src3[...].reshape(1, BE)
    lanes0 = jax.lax.broadcasted_iota(jnp.int32, (128, BE), 0)

    def win(w, _):
        m = (lanes0 + w * 128) == srcrow
        contrib = _dot(m.astype(jnp.float32), e)
        base = w * 128
        tot[pl.ds(base, 128), :] += contrib
        return _

    lax.fori_loop(wlo[i], whi[i] + 1, win, 0)


def _apply_body(cells, tot_ref, Wc1, bc1, Wc2, bc2, Wc3, bc3,
                We1, be1, We2, be2, We3, be3,
                Wp1c, wp1t, wp1d, bp1, Wp2, bp2, Wp3, bp3, pred):
    x = cells[...]
    tot = tot_ref[...]
    h = _relu(_dot(x, Wc1[...]) + bc1[...])
    h = _relu(_dot(h, Wc2[...]) + bc2[...])
    adc = _dot(h, Wc3[...]) + bc3[...]
    h = _relu(_dot(tot, We1[...]) + be1[...])
    h = _relu(_dot(h, We2[...]) + be2[...])
    ade = _dot(h, We3[...]) + be3[...]
    g = _relu(_dot(x, Wp1c[...]) + _dot(tot, wp1t[...])
              + _dot(adc * ade, wp1d[...]) + bp1[...])
    g = _relu(_dot(g, Wp2[...]) + bp2[...])
    pred[...] = _dot(g, Wp3[...]) + bp3[...]


def kernel(grid_obs, effect_inds, params):
    cells = grid_obs
    src = effect_inds[0].astype(jnp.int32)
    dst = effect_inds[1].astype(jnp.int32)
    f32 = jnp.float32

    (We1, be1), (We2, be2), (We3, be3) = params['effect']
    Wa, Wb, wc = We1[:OBS], We1[OBS:2 * OBS], We1[2 * OBS]
    edc, edn = params['effect_dotp_cell'], params['effect_dotp_neighbor']
    Wab = jnp.concatenate([Wa, Wb], axis=1)
    Wd1 = jnp.concatenate([edc[0][0], edn[0][0]], axis=1)
    bd1 = jnp.concatenate([edc[0][1], edn[0][1]])[None]
    Wd2 = jnp.zeros((40, 40), f32).at[:20, :20].set(edc[1][0]).at[20:, 20:].set(edn[1][0])
    bd2 = jnp.concatenate([edc[1][1], edn[1][1]])[None]
    Wd3 = jnp.zeros((40, 2), f32).at[:20, 0:1].set(edc[2][0]).at[20:, 1:2].set(edn[2][0])
    bd3 = jnp.concatenate([edc[2][1], edn[2][1]])[None]
    eye20 = jnp.eye(20, dtype=f32)
    S1s = jnp.zeros((40, TW), f32).at[:20, :20].set(eye20)
    S1d = jnp.zeros((40, TW), f32).at[20:, :20].set(eye20)
    S2s = jnp.zeros((2, TW), f32).at[0, 20].set(1.0)
    S2d = jnp.zeros((2, TW), f32).at[1, 21].set(1.0)

    tabS, tabD = pl.pallas_call(
        _tables_body,
        out_shape=(jax.ShapeDtypeStruct((N_CELLS, TW), f32),
                   jax.ShapeDtypeStruct((N_CELLS, TW), f32)),
    )(cells, Wab, Wd1, bd1, Wd2, bd2, Wd3, bd3, S1s, S2s, S1d, S2d)

    mesh = plsc.VectorSubcoreMesh(core_axis_name="c", subcore_axis_name="s")
    gsum = pl.kernel(
        _sc_gather_body,
        out_type=jax.ShapeDtypeStruct((N_EDGES, TW), f32),
        mesh=mesh,
        scratch_types=[pltpu.VMEM((CH,), jnp.int32),
                       pltpu.VMEM((CH,), jnp.int32),
                       pltpu.VMEM((CH, TW), f32),
                       pltpu.VMEM((CH, TW), f32),
                       pltpu.SemaphoreType.DMA,
                       pltpu.SemaphoreType.DMA],
        compiler_params=pltpu.CompilerParams(use_tc_tiling_on_sc=False),
    )(tabS, tabD, src, dst)

    s20 = jnp.zeros((TW, 1), f32).at[20, 0].set(1.0)
    s21 = jnp.zeros((TW, 1), f32).at[21, 0].set(1.0)
    wc32 = jnp.zeros((1, TW), f32).at[0, :20].set(wc)
    b1 = jnp.zeros((1, TW), f32).at[0, :20].set(be1)
    W2p = jnp.zeros((TW, TW), f32).at[:20, :20].set(We2)
    b2 = jnp.zeros((1, TW), f32).at[0, :20].set(be2)
    w3p = jnp.zeros((TW, 1), f32).at[:20].set(We3)
    b3 = be3[None]

    def _w(a):
        return pl.BlockSpec(a.shape, lambda i: (0,) * a.ndim)

    src2 = src.reshape(N_EDGES // BE, BE)
    wlo = src2[:, 0] // 128
    whi = src2[:, -1] // 128
    src3 = src.reshape(N_EDGES // BE, 1, BE)
    smem = pl.BlockSpec(memory_space=pltpu.SMEM)
    tot_pad = pl.pallas_call(
        _edge_body,
        grid=(N_EDGES // BE,),
        in_specs=[smem, smem,
                  pl.BlockSpec((BE, TW), lambda i: (i, 0)),
                  pl.BlockSpec((1, 1, BE), lambda i: (i, 0, 0)),
                  _w(s20), _w(s21), _w(wc32), _w(b1), _w(W2p), _w(b2),
                  _w(w3p), _w(b3)],
        out_specs=pl.BlockSpec((NWIN * 128, 1), lambda i: (0, 0)),
        out_shape=jax.ShapeDtypeStruct((NWIN * 128, 1), f32),
    )(wlo, whi, gsum, src3, s20, s21, wc32, b1, W2p, b2, w3p, b3)
    tot = tot_pad[:N_CELLS]

    adc_p, ade_p, app_p = (params['apply_dotp_cell'],
                           params['apply_dotp_effect'], params['apply'])
    (Wp1, bp1), (Wp2, bp2), (Wp3, bp3) = app_p
    pred = pl.pallas_call(
        _apply_body,
        out_shape=jax.ShapeDtypeStruct((N_CELLS, OBS), f32),
    )(cells, tot,
      adc_p[0][0], adc_p[0][1][None], adc_p[1][0], adc_p[1][1][None],
      adc_p[2][0], adc_p[2][1][None],
      ade_p[0][0], ade_p[0][1][None], ade_p[1][0], ade_p[1][1][None],
      ade_p[2][0], ade_p[2][1][None],
      Wp1[:OBS], Wp1[OBS:OBS + 1], Wp1[OBS + 1:OBS + 2], bp1[None],
      Wp2, bp2[None], Wp3, bp3[None])

    return jnp.stack([pred, pred])
